# baseline (speedup 1.0000x reference)
import functools

import jax
import jax.numpy as jnp
from jax import lax
from jax.experimental import pallas as pl
from jax.experimental.pallas import tpu as pltpu
from jax.experimental.pallas import tpu_sc as plsc

_K = 32
_NC, _NS = 2, 16
_NW = _NC * _NS
_RB = 8
_HB = 272


def _row_threshold_bits(x_v, hist_v, cand_v, roff, mc):
    ones = jnp.ones((16,), jnp.int32)
    zeros = jnp.zeros((16,), jnp.int32)

    for c in range(_HB // 16):
        hist_v[pl.ds(c * 16, 16)] = zeros

    def pass_a(c, carry):
        v = x_v[pl.ds(roff + c * 16, 16)]
        b = plsc.bitcast(v, jnp.int32)
        e = jnp.maximum(lax.shift_right_arithmetic(b, 23), 0)
        plsc.addupdate_scatter(hist_v, [e], ones)
        return carry

    lax.fori_loop(0, mc, pass_a, 0)

    def scan_bins(i, carry):
        run, nge = carry
        c = (_HB // 16 - 1) - i
        h = hist_v[pl.ds(c * 16, 16)]
        suf = plsc.cumsum(jnp.flip(h, 0)) + run
        nge = nge + jnp.sum((suf >= _K).astype(jnp.int32))
        run = run + jnp.sum(h)
        return run, nge

    _, nge = lax.fori_loop(0, _HB // 16, scan_bins, (0, 0))
    e_bound = jnp.maximum(nge - 1, 0)

    def pass_b(c, off):
        v = x_v[pl.ds(roff + c * 16, 16)]
        b = plsc.bitcast(v, jnp.int32)
        e = jnp.maximum(lax.shift_right_arithmetic(b, 23), 0)
        mk = e >= e_bound
        plsc.store_compressed(cand_v.at[pl.ds(off, 16)], b, mask=mk)
        return off + jnp.sum(mk.astype(jnp.int32))

    noff = lax.fori_loop(0, mc, pass_b, 0)
    cand_v[pl.ds(noff, 16)] = zeros

    nc = (noff + 15) // 16
    t = e_bound << 23

    def count_ge(trial):
        def cb(c, acc):
            cb_bits = cand_v[pl.ds(c * 16, 16)]
            return acc + (cb_bits >= trial).astype(jnp.int32)

        accv = lax.fori_loop(0, nc, cb, zeros)
        return jnp.sum(accv)

    for bit in range(22, -1, -1):
        trial = t | (1 << bit)
        t = jnp.where(count_ge(trial) >= _K, trial, t)
    return t


def _sc_body(a_hbm, o_hbm, x_v, hist_v, cand_v, *, n, m, nbatch):
    mc = m // 16
    wid = lax.axis_index("s") * _NC + lax.axis_index("c")
    nbase, nrem = n // _NW, n % _NW
    n_w = nbase + jnp.where(wid < nrem, 1, 0)
    start = wid * nbase + jnp.minimum(wid, nrem)
    end = start + n_w

    def batch(j, carry):
        bstart = jnp.minimum(start + j * _RB, end - _RB)
        pltpu.sync_copy(a_hbm.at[pl.ds(bstart * m, _RB * m)], x_v)
        for rr in range(_RB):
            roff = rr * m
            t = _row_threshold_bits(x_v, hist_v, cand_v, roff, mc)

            def pass_c(c, carry):
                sl = pl.ds(roff + c * 16, 16)
                v = x_v[sl]
                b = plsc.bitcast(v, jnp.int32)
                x_v[sl] = jnp.where(b >= t, v, 0.0)
                return carry

            lax.fori_loop(0, mc, pass_c, 0)
        pltpu.sync_copy(x_v, o_hbm.at[pl.ds(bstart * m, _RB * m)])
        return carry

    lax.fori_loop(0, nbatch, batch, 0)


@functools.cache
def _make_sc_kernel(n, m):
    assert m % 16 == 0 and n % _NW >= 0
    nbatch = (n // _NW + (1 if n % _NW else 0) + _RB - 1) // _RB
    mesh = plsc.VectorSubcoreMesh(core_axis_name="c", subcore_axis_name="s")
    return pl.kernel(
        functools.partial(_sc_body, n=n, m=m, nbatch=nbatch),
        out_type=jax.ShapeDtypeStruct((n * m,), jnp.float32),
        mesh=mesh,
        compiler_params=pltpu.CompilerParams(needs_layout_passes=False),
        scratch_types=[
            pltpu.VMEM((_RB * m,), jnp.float32),
            pltpu.VMEM((_HB,), jnp.int32),
            pltpu.VMEM((m + 32,), jnp.int32),
        ],
    )


@jax.jit
def _run(A):
    n, m = A.shape
    out = _make_sc_kernel(n, m)(A.reshape(-1))
    return out.reshape(n, m)


def kernel(idx, A):
    return _run(A)


# speedup vs baseline: 1.0538x; 1.0538x over previous
import functools

import jax
import jax.numpy as jnp
from jax import lax
from jax.experimental import pallas as pl
from jax.experimental.pallas import tpu as pltpu
from jax.experimental.pallas import tpu_sc as plsc

_K = 32
_NC, _NS = 2, 16
_NW = _NC * _NS
_RB = 8
_HB = 272


def _row_threshold_bits(x_v, hist_v, cand_v, roff, mc):
    ones = jnp.ones((16,), jnp.int32)
    zeros = jnp.zeros((16,), jnp.int32)

    for c in range(_HB // 16):
        hist_v[pl.ds(c * 16, 16)] = zeros

    def pass_a(c, carry):
        v = x_v[pl.ds(roff + c * 16, 16)]
        b = plsc.bitcast(v, jnp.int32)
        e = jnp.maximum(lax.shift_right_arithmetic(b, 23), 0)
        plsc.addupdate_scatter(hist_v, [e], ones)
        return carry

    lax.fori_loop(0, mc, pass_a, 0, unroll=8)

    def scan_bins(i, carry):
        run, nge = carry
        c = (_HB // 16 - 1) - i
        h = hist_v[pl.ds(c * 16, 16)]
        suf = plsc.cumsum(jnp.flip(h, 0)) + run
        nge = nge + jnp.sum((suf >= _K).astype(jnp.int32))
        run = run + jnp.sum(h)
        return run, nge

    _, nge = lax.fori_loop(0, _HB // 16, scan_bins, (0, 0))
    e_bound = jnp.maximum(nge - 1, 0)

    def pass_b(c, off):
        v = x_v[pl.ds(roff + c * 16, 16)]
        b = plsc.bitcast(v, jnp.int32)
        e = jnp.maximum(lax.shift_right_arithmetic(b, 23), 0)
        mk = e >= e_bound
        plsc.store_compressed(cand_v.at[pl.ds(off, 16)], b, mask=mk)
        return off + jnp.sum(mk.astype(jnp.int32))

    noff = lax.fori_loop(0, mc, pass_b, 0, unroll=8)
    cand_v[pl.ds(noff, 16)] = zeros

    nc = (noff + 15) // 16
    t = e_bound << 23

    def count_ge(trial):
        def cb(c, acc):
            cb_bits = cand_v[pl.ds(c * 16, 16)]
            return acc + (cb_bits >= trial).astype(jnp.int32)

        accv = lax.fori_loop(0, nc, cb, zeros)
        return jnp.sum(accv)

    for bit in range(22, -1, -1):
        trial = t | (1 << bit)
        t = jnp.where(count_ge(trial) >= _K, trial, t)
    return t


def _sc_body(a_hbm, o_hbm, x_v, hist_v, cand_v, *, n, m, nbatch):
    mc = m // 16
    wid = lax.axis_index("s") * _NC + lax.axis_index("c")
    nbase, nrem = n // _NW, n % _NW
    n_w = nbase + jnp.where(wid < nrem, 1, 0)
    start = wid * nbase + jnp.minimum(wid, nrem)
    end = start + n_w

    def batch(j, carry):
        bstart = jnp.minimum(start + j * _RB, end - _RB)
        pltpu.sync_copy(a_hbm.at[pl.ds(bstart * m, _RB * m)], x_v)
        for rr in range(_RB):
            roff = rr * m
            t = _row_threshold_bits(x_v, hist_v, cand_v, roff, mc)

            def pass_c(c, carry):
                sl = pl.ds(roff + c * 16, 16)
                v = x_v[sl]
                b = plsc.bitcast(v, jnp.int32)
                x_v[sl] = jnp.where(b >= t, v, 0.0)
                return carry

            lax.fori_loop(0, mc, pass_c, 0, unroll=8)
        pltpu.sync_copy(x_v, o_hbm.at[pl.ds(bstart * m, _RB * m)])
        return carry

    lax.fori_loop(0, nbatch, batch, 0)


@functools.cache
def _make_sc_kernel(n, m):
    assert m % 16 == 0 and n % _NW >= 0
    nbatch = (n // _NW + (1 if n % _NW else 0) + _RB - 1) // _RB
    mesh = plsc.VectorSubcoreMesh(core_axis_name="c", subcore_axis_name="s")
    return pl.kernel(
        functools.partial(_sc_body, n=n, m=m, nbatch=nbatch),
        out_type=jax.ShapeDtypeStruct((n * m,), jnp.float32),
        mesh=mesh,
        compiler_params=pltpu.CompilerParams(needs_layout_passes=False),
        scratch_types=[
            pltpu.VMEM((_RB * m,), jnp.float32),
            pltpu.VMEM((_HB,), jnp.int32),
            pltpu.VMEM((m + 32,), jnp.int32),
        ],
    )


@jax.jit
def _run(A):
    n, m = A.shape
    out = _make_sc_kernel(n, m)(A.reshape(-1))
    return out.reshape(n, m)


def kernel(idx, A):
    return _run(A)


# speedup vs baseline: 1.7784x; 1.6875x over previous
import functools

import jax
import jax.numpy as jnp
from jax import lax
from jax.experimental import pallas as pl
from jax.experimental.pallas import tpu as pltpu
from jax.experimental.pallas import tpu_sc as plsc

_K = 32
_NC, _NS = 2, 16
_NW = _NC * _NS
_RB = 8
_MARGIN = 1 << 21


def _collect(x_v, cand_v, roff, mc, thr, unroll):
    def cc(c, off):
        v = x_v[pl.ds(roff + c * 16, 16)]
        b = plsc.bitcast(v, jnp.int32)
        mk = b >= thr
        mki = mk.astype(jnp.int32)
        pos = plsc.cumsum(mki)
        plsc.store_scatter(cand_v, [pos + (off - 1)], b, mask=mk)
        return off + plsc.all_reduce_population_count(mk)

    return cc


def _row_threshold_bits(x_v, cand_v, roff, mc, thr_pred):
    zeros = jnp.zeros((16,), jnp.int32)

    off = lax.fori_loop(0, mc, _collect(x_v, cand_v, roff, mc, thr_pred, 8),
                        zeros, unroll=8)
    noff = jnp.max(off)

    nfb = jnp.where(noff < _K, mc, 0)
    off2 = lax.fori_loop(0, nfb, _collect(x_v, cand_v, roff, mc, 0, 1),
                         zeros)
    noff = jnp.where(noff < _K, jnp.max(off2), noff)

    for k in range(4):
        cand_v[pl.ds(noff + 16 * k, 16)] = zeros
    nc4 = (noff + 63) // 64

    t = 0

    def count_ge(trial):
        def cb(c, acc):
            for k in range(4):
                cbits = cand_v[pl.ds((c * 4 + k) * 16, 16)]
                acc = acc + (cbits >= trial).astype(jnp.int32)
            return acc

        return jnp.sum(lax.fori_loop(0, nc4, cb, zeros))

    for bit in range(30, -1, -1):
        trial = t | (1 << bit)
        t = jnp.where(count_ge(trial) >= _K, trial, t)
    return t


def _sc_body(a_hbm, o_hbm, x_v, cand_v, *, n, m, nbatch):
    mc = m // 16
    wid = lax.axis_index("s") * _NC + lax.axis_index("c")
    nbase, nrem = n // _NW, n % _NW
    n_w = nbase + jnp.where(wid < nrem, 1, 0)
    start = wid * nbase + jnp.minimum(wid, nrem)
    end = start + n_w

    def batch(j, thr_pred):
        bstart = jnp.minimum(start + j * _RB, end - _RB)
        pltpu.sync_copy(a_hbm.at[pl.ds(bstart * m, _RB * m)], x_v)
        for rr in range(_RB):
            roff = rr * m
            t = _row_threshold_bits(x_v, cand_v, roff, mc, thr_pred)

            def pass_c(c, carry):
                sl = pl.ds(roff + c * 16, 16)
                v = x_v[sl]
                b = plsc.bitcast(v, jnp.int32)
                x_v[sl] = jnp.where(b >= t, v, 0.0)
                return carry

            lax.fori_loop(0, mc, pass_c, 0, unroll=8)
            thr_pred = jnp.maximum(t - _MARGIN, 0)
        pltpu.sync_copy(x_v, o_hbm.at[pl.ds(bstart * m, _RB * m)])
        return thr_pred

    lax.fori_loop(0, nbatch, batch, 0)


@functools.cache
def _make_sc_kernel(n, m):
    assert m % 16 == 0
    nbatch = (n // _NW + (1 if n % _NW else 0) + _RB - 1) // _RB
    mesh = plsc.VectorSubcoreMesh(core_axis_name="c", subcore_axis_name="s")
    return pl.kernel(
        functools.partial(_sc_body, n=n, m=m, nbatch=nbatch),
        out_type=jax.ShapeDtypeStruct((n * m,), jnp.float32),
        mesh=mesh,
        compiler_params=pltpu.CompilerParams(needs_layout_passes=False),
        scratch_types=[
            pltpu.VMEM((_RB * m,), jnp.float32),
            pltpu.VMEM((m + 128,), jnp.int32),
        ],
    )


@jax.jit
def _run(A):
    n, m = A.shape
    out = _make_sc_kernel(n, m)(A.reshape(-1))
    return out.reshape(n, m)


def kernel(idx, A):
    return _run(A)


# speedup vs baseline: 3.7961x; 2.1346x over previous
import functools

import jax
import jax.numpy as jnp
from jax import lax
from jax.experimental import pallas as pl
from jax.experimental.pallas import tpu as pltpu
from jax.experimental.pallas import tpu_sc as plsc

_K = 32
_NC, _NS = 2, 16
_NW = _NC * _NS
_RB = 8
_MARGIN = 1 << 21


def _collect(x_v, cand_v, roff, thr):
    def cc(c, off):
        v = x_v[pl.ds(roff + c * 16, 16)]
        b = plsc.bitcast(v, jnp.int32)
        mk = b >= thr
        mki = mk.astype(jnp.int32)
        pos = plsc.cumsum(mki)
        plsc.store_scatter(cand_v, [pos + (off - 1)], b, mask=mk)
        return off + plsc.all_reduce_population_count(mk)

    return cc


def _row_threshold_bits(x_v, cand_v, roff, mc, thr_pred):
    zeros = jnp.zeros((16,), jnp.int32)

    off = plsc.parallel_loop(0, mc, unroll=8, carry=zeros)(
        _collect(x_v, cand_v, roff, thr_pred))
    noff = jnp.max(off)

    nfb = jnp.where(noff < _K, mc, 0)
    off2 = plsc.parallel_loop(0, nfb, carry=zeros)(
        _collect(x_v, cand_v, roff, 0))
    noff = jnp.where(noff < _K, jnp.max(off2), noff)

    for k in range(4):
        cand_v[pl.ds(noff + 16 * k, 16)] = zeros
    nc4 = (noff + 63) // 64

    t = 0

    def count_ge(trial):
        def cb(c, acc):
            for k in range(4):
                cbits = cand_v[pl.ds((c * 4 + k) * 16, 16)]
                acc = acc + (cbits >= trial).astype(jnp.int32)
            return acc

        return jnp.sum(plsc.parallel_loop(0, nc4, carry=zeros)(cb))

    for bit in range(30, -1, -1):
        trial = t | (1 << bit)
        t = jnp.where(count_ge(trial) >= _K, trial, t)
    return t


def _sc_body(a_hbm, o_hbm, x_v, cand_v, *, n, m, nbatch):
    mc = m // 16
    wid = lax.axis_index("s") * _NC + lax.axis_index("c")
    nbase, nrem = n // _NW, n % _NW
    n_w = nbase + jnp.where(wid < nrem, 1, 0)
    start = wid * nbase + jnp.minimum(wid, nrem)
    end = start + n_w

    def batch(j, thr_pred):
        bstart = jnp.minimum(start + j * _RB, end - _RB)
        pltpu.sync_copy(a_hbm.at[pl.ds(bstart * m, _RB * m)], x_v)
        for rr in range(_RB):
            roff = rr * m
            t = _row_threshold_bits(x_v, cand_v, roff, mc, thr_pred)

            @plsc.parallel_loop(0, mc, unroll=8)
            def pass_c(c):
                sl = pl.ds(roff + c * 16, 16)
                v = x_v[sl]
                b = plsc.bitcast(v, jnp.int32)
                x_v[sl] = jnp.where(b >= t, v, 0.0)
            thr_pred = jnp.maximum(t - _MARGIN, 0)
        pltpu.sync_copy(x_v, o_hbm.at[pl.ds(bstart * m, _RB * m)])
        return thr_pred

    lax.fori_loop(0, nbatch, batch, 0)


@functools.cache
def _make_sc_kernel(n, m):
    assert m % 16 == 0
    nbatch = (n // _NW + (1 if n % _NW else 0) + _RB - 1) // _RB
    mesh = plsc.VectorSubcoreMesh(core_axis_name="c", subcore_axis_name="s")
    return pl.kernel(
        functools.partial(_sc_body, n=n, m=m, nbatch=nbatch),
        out_type=jax.ShapeDtypeStruct((n * m,), jnp.float32),
        mesh=mesh,
        compiler_params=pltpu.CompilerParams(needs_layout_passes=False),
        scratch_types=[
            pltpu.VMEM((_RB * m,), jnp.float32),
            pltpu.VMEM((m + 128,), jnp.int32),
        ],
    )


@jax.jit
def _run(A):
    n, m = A.shape
    out = _make_sc_kernel(n, m)(A.reshape(-1))
    return out.reshape(n, m)


def kernel(idx, A):
    return _run(A)


# speedup vs baseline: 4.5052x; 1.1868x over previous
import functools

import jax
import jax.numpy as jnp
from jax import lax
from jax.experimental import pallas as pl
from jax.experimental.pallas import tpu as pltpu
from jax.experimental.pallas import tpu_sc as plsc

_K = 32
_NC, _NS = 2, 16
_NW = _NC * _NS
_RB = 2
_MARGIN = 1 << 20
_SC = 8


def _splat_last(x):
    return lax.gather(
        x,
        jnp.full((16, 1), 15, jnp.int32),
        lax.GatherDimensionNumbers(
            offset_dims=(), collapsed_slice_dims=(0,), start_index_map=(0,)),
        (1,),
        mode=lax.GatherScatterMode.PROMISE_IN_BOUNDS,
    )


def _collect(x_v, cand_v, roff, thr):
    def cc(c, off):
        v = x_v[pl.ds(roff + c * 16, 16)]
        b = plsc.bitcast(v, jnp.int32)
        mk = b >= thr
        mki = mk.astype(jnp.int32)
        pos = plsc.cumsum(mki)
        plsc.store_scatter(cand_v, [pos + (off - 1)], b, mask=mk)
        return off + plsc.all_reduce_population_count(mk)

    return cc


def _row_threshold_bits(x_v, cand_v, roff, mc, thr_pred):
    zeros = jnp.zeros((16,), jnp.int32)

    for k in range(_SC + 1):
        cand_v[pl.ds(k * 16, 16)] = zeros

    off = plsc.parallel_loop(0, mc, unroll=8, carry=zeros)(
        _collect(x_v, cand_v, roff, thr_pred))
    noff = jnp.max(off)

    nfb = jnp.where(noff < _K, mc, 0)
    off2 = plsc.parallel_loop(0, nfb, carry=zeros)(
        _collect(x_v, cand_v, roff, 0))
    noff = jnp.where(noff < _K, jnp.max(off2), noff)

    def static_path():
        cands = [cand_v[pl.ds(k * 16, 16)] for k in range(_SC)]
        t = zeros
        for bit in range(30, -1, -1):
            trial = t | (1 << bit)
            acc = zeros
            for ck in cands:
                acc = acc + (ck >= trial).astype(jnp.int32)
            cnt = _splat_last(plsc.cumsum(acc))
            t = jnp.where(cnt >= _K, trial, t)
        return t

    def dynamic_path():
        for k in range(4):
            cand_v[pl.ds(noff + 16 * k, 16)] = zeros
        nc4 = (noff + 63) // 64
        t = 0

        def count_ge(trial):
            def cb(c, acc):
                for k in range(4):
                    cbits = cand_v[pl.ds((c * 4 + k) * 16, 16)]
                    acc = acc + (cbits >= trial).astype(jnp.int32)
                return acc

            return jnp.sum(plsc.parallel_loop(0, nc4, carry=zeros)(cb))

        for bit in range(30, -1, -1):
            trial = t | (1 << bit)
            t = jnp.where(count_ge(trial) >= _K, trial, t)
        return jnp.full((16,), t, jnp.int32)

    return lax.cond(noff <= 16 * _SC, static_path, dynamic_path)


def _sc_body(a_hbm, o_hbm, x0, x1, cand_v, si0, si1, so0, so1,
             *, n, m, nbatch):
    mc = m // 16
    wid = lax.axis_index("s") * _NC + lax.axis_index("c")
    nbase, nrem = n // _NW, n % _NW
    n_w = nbase + jnp.where(wid < nrem, 1, 0)
    start = wid * nbase + jnp.minimum(wid, nrem)
    end = start + n_w

    def bslice(j):
        bs = jnp.minimum(start + j * _RB, end - _RB)
        return pl.ds(bs * m, _RB * m)

    def do_batch(j, xc, sic, soc, xn, son_other, sin_next, thr_pred):
        @pl.when(j >= 1)
        def _():
            pltpu.make_async_copy(xn, o_hbm.at[bslice(j)], son_other).wait()

        pltpu.async_copy(a_hbm.at[bslice(j + 1)], xn, sin_next)
        pltpu.make_async_copy(a_hbm.at[bslice(j)], xc, sic).wait()

        for rr in range(_RB):
            roff = rr * m
            t = _row_threshold_bits(x_v=xc, cand_v=cand_v, roff=roff,
                                    mc=mc, thr_pred=thr_pred)

            @plsc.parallel_loop(0, mc, unroll=8)
            def pass_c(c):
                sl = pl.ds(roff + c * 16, 16)
                v = xc[sl]
                b = plsc.bitcast(v, jnp.int32)
                xc[sl] = jnp.where(b >= t, v, 0.0)

            thr_pred = jnp.maximum(t - _MARGIN, 0)
        pltpu.async_copy(xc, o_hbm.at[bslice(j)], soc)
        return thr_pred

    pltpu.async_copy(a_hbm.at[bslice(0)], x0, si0)

    def pair(i, thr_pred):
        thr_pred = do_batch(2 * i, x0, si0, so0, x1, so1, si1, thr_pred)
        thr_pred = do_batch(2 * i + 1, x1, si1, so1, x0, so0, si0, thr_pred)
        return thr_pred

    nb2 = (nbatch + 1) // 2
    lax.fori_loop(0, nb2, pair, jnp.zeros((16,), jnp.int32))

    pltpu.make_async_copy(x1, o_hbm.at[bslice(2 * nb2 - 1)], so1).wait()
    pltpu.make_async_copy(a_hbm.at[bslice(2 * nb2)], x0, si0).wait()


@functools.cache
def _make_sc_kernel(n, m):
    assert m % 16 == 0
    nbatch = (n // _NW + (1 if n % _NW else 0) + _RB - 1) // _RB
    mesh = plsc.VectorSubcoreMesh(core_axis_name="c", subcore_axis_name="s")
    return pl.kernel(
        functools.partial(_sc_body, n=n, m=m, nbatch=nbatch),
        out_type=jax.ShapeDtypeStruct((n * m,), jnp.float32),
        mesh=mesh,
        compiler_params=pltpu.CompilerParams(needs_layout_passes=False),
        scratch_types=[
            pltpu.VMEM((_RB * m,), jnp.float32),
            pltpu.VMEM((_RB * m,), jnp.float32),
            pltpu.VMEM((m + 160,), jnp.int32),
            pltpu.SemaphoreType.DMA,
            pltpu.SemaphoreType.DMA,
            pltpu.SemaphoreType.DMA,
            pltpu.SemaphoreType.DMA,
        ],
    )


@jax.jit
def _run(A):
    n, m = A.shape
    out = _make_sc_kernel(n, m)(A.reshape(-1))
    return out.reshape(n, m)


def kernel(idx, A):
    return _run(A)
